# baseline (speedup 1.0000x reference)
import functools

import jax
import jax.numpy as jnp
from jax import lax
from jax.experimental import pallas as pl
from jax.experimental.pallas import tpu as pltpu
from jax.experimental.pallas import tpu_sc as plsc

_NC = 2
_NS = 16
_NW = _NC * _NS

_N = 10000
_NPAD = 10240
_RPT = _NPAD // _NS
_F = 64
_E = 320000
_EPW = _E // _NW
_C = 80
_NCH = _EPW // _C
_B = 64
_NCLS = 10

_mesh = plsc.VectorSubcoreMesh(core_axis_name="c", subcore_axis_name="s")
_cp = pltpu.CompilerParams(use_tc_tiling_on_sc=False)


_NBUF = 5


@functools.partial(
    pl.kernel,
    out_type=jax.ShapeDtypeStruct((_NC * _NPAD,), jnp.float32),
    mesh=_mesh,
    scratch_types=[
        pltpu.VMEM((_NBUF, _C), jnp.int32),
        pltpu.VMEM((_C,), jnp.float32),
        pltpu.VMEM_SHARED((_NPAD,), jnp.float32),
        pltpu.SemaphoreType.DMA((_NBUF,)),
        pltpu.SemaphoreType.DMA((_NBUF,)),
    ],
    compiler_params=_cp,
)
def _sc_degree(dst_hbm, zeros1_hbm, out_hbm, didx, ones_v, accum, isem, ssem):
    cid = lax.axis_index("c")
    sid = lax.axis_index("s")
    wid = cid * _NS + sid
    base = wid * _EPW

    @pl.loop(0, _C, step=16)
    def _(i):
        ones_v[pl.ds(i, 16)] = jnp.full((16,), 1.0, jnp.float32)

    pltpu.sync_copy(zeros1_hbm.at[pl.ds(sid * _RPT, _RPT)],
                    accum.at[pl.ds(sid * _RPT, _RPT)])
    plsc.subcore_barrier()

    def start_idx(j, b):
        pltpu.async_copy(dst_hbm.at[pl.ds(base + j * _C, _C)], didx.at[b],
                         isem.at[b])

    def wait_idx(j, b):
        pltpu.make_async_copy(dst_hbm.at[pl.ds(base + j * _C, _C)], didx.at[b],
                              isem.at[b]).wait()

    def start_scat(b):
        pltpu.async_copy(ones_v, accum.at[didx.at[b]], ssem.at[b], add=True)

    def wait_scat(b):
        pltpu.make_async_copy(ones_v, accum.at[didx.at[b]], ssem.at[b]).wait()

    for b in range(_NBUF):
        start_idx(b, b)

    @pl.loop(0, _NCH - _NBUF, step=_NBUF)
    def _(j0):
        for b in range(_NBUF):
            wait_idx(j0 + b, b)
            start_scat(b)
        for b in range(_NBUF):
            wait_scat(b)
            start_idx(j0 + _NBUF + b, b)

    for b in range(_NBUF):
        wait_idx(_NCH - _NBUF + b, b)
        start_scat(b)
    for b in range(_NBUF):
        wait_scat(b)

    plsc.subcore_barrier()
    pltpu.sync_copy(accum.at[pl.ds(sid * _RPT, _RPT)],
                    out_hbm.at[pl.ds(cid * _NPAD + sid * _RPT, _RPT)])


@functools.partial(
    pl.kernel,
    out_type=jax.ShapeDtypeStruct((_NC * _NPAD, _F), jnp.float32),
    mesh=_mesh,
    scratch_types=[
        pltpu.VMEM((_EPW,), jnp.int32),
        pltpu.VMEM((_NBUF, _C), jnp.int32),
        pltpu.VMEM((_NBUF, _C, _F), jnp.float32),
        pltpu.VMEM_SHARED((_NPAD, _F), jnp.float32),
        pltpu.SemaphoreType.DMA((_NBUF,)),
        pltpu.SemaphoreType.DMA((_NBUF,)),
        pltpu.SemaphoreType.DMA((_NBUF,)),
    ],
    compiler_params=_cp,
)
def _sc_scatter(hp_hbm, src_hbm, dst_hbm, zeros2_hbm, out_hbm,
                sidx, didx, rows, accum, gsem, isem, ssem):
    cid = lax.axis_index("c")
    sid = lax.axis_index("s")
    wid = cid * _NS + sid
    base = wid * _EPW

    pltpu.sync_copy(src_hbm.at[pl.ds(base, _EPW)], sidx)
    pltpu.sync_copy(zeros2_hbm.at[pl.ds(sid * _RPT, _RPT)],
                    accum.at[pl.ds(sid * _RPT, _RPT)])
    plsc.subcore_barrier()

    def start_idx(j, b):
        pltpu.async_copy(dst_hbm.at[pl.ds(base + j * _C, _C)], didx.at[b],
                         isem.at[b])

    def wait_idx(j, b):
        pltpu.make_async_copy(dst_hbm.at[pl.ds(base + j * _C, _C)], didx.at[b],
                              isem.at[b]).wait()

    def start_gat(j, b):
        pltpu.async_copy(hp_hbm.at[sidx.at[pl.ds(j * _C, _C)]], rows.at[b],
                         gsem.at[b])

    def wait_gat(j, b):
        pltpu.make_async_copy(hp_hbm.at[sidx.at[pl.ds(j * _C, _C)]],
                              rows.at[b], gsem.at[b]).wait()

    def start_scat(b):
        pltpu.async_copy(rows.at[b], accum.at[didx.at[b]], ssem.at[b],
                         add=True)

    def wait_scat(b):
        pltpu.make_async_copy(rows.at[b], accum.at[didx.at[b]],
                              ssem.at[b]).wait()

    for b in range(_NBUF):
        start_idx(b, b)
        start_gat(b, b)

    @pl.loop(0, _NCH - _NBUF, step=_NBUF)
    def _(j0):
        for b in range(_NBUF):
            wait_idx(j0 + b, b)
            wait_gat(j0 + b, b)
            start_scat(b)
        for b in range(_NBUF):
            wait_scat(b)
            start_idx(j0 + _NBUF + b, b)
            start_gat(j0 + _NBUF + b, b)

    for b in range(_NBUF):
        wait_idx(_NCH - _NBUF + b, b)
        wait_gat(_NCH - _NBUF + b, b)
        start_scat(b)
    for b in range(_NBUF):
        wait_scat(b)

    plsc.subcore_barrier()
    pltpu.sync_copy(accum.at[pl.ds(sid * _RPT, _RPT)],
                    out_hbm.at[pl.ds(cid * _NPAD + sid * _RPT, _RPT)])


_NH = _N // 2
_HPAD = _NPAD // 2


def _edge_body(e_hbm, s_hbm, d_hbm, sem0, sem1):
    c0 = pltpu.make_async_copy(e_hbm.at[0], s_hbm, sem0)
    c1 = pltpu.make_async_copy(e_hbm.at[1], d_hbm, sem1)
    c0.start()
    c1.start()
    c0.wait()
    c1.wait()


def _mm_body(x_ref, w_ref, o_ref):
    o_ref[...] = jnp.dot(x_ref[...], w_ref[...],
                         preferred_element_type=jnp.float32)


def _dfull(dinv2_ref):
    d0 = jnp.broadcast_to(dinv2_ref[:, 0:1], (_NH, _F))
    d1 = jnp.broadcast_to(dinv2_ref[:, 1:2], (_NH, _F))
    return jnp.concatenate([d0, d1], axis=1)


def _scale_body(h_ref, dinv2_ref, o_ref):
    o_ref[...] = _dfull(dinv2_ref) * h_ref[...]


def _mid_body(s_ref, hp_ref, dinv2_ref, w2_ref, b1_ref, h2p_ref):
    dinv = _dfull(dinv2_ref)
    ssum = s_ref[:_NH, :] + s_ref[_HPAD:_HPAD + _NH, :]
    z = jnp.maximum(dinv * (ssum + hp_ref[...]) + b1_ref[...], 0.0)
    h2 = jnp.dot(z, w2_ref[...], preferred_element_type=jnp.float32)
    h2p_ref[...] = dinv * h2


def _final_body(s_ref, hp_ref, dinv2_ref, b2_ref, batch_ref,
                wfc_ref, bfc_ref, o_ref):
    dinv = _dfull(dinv2_ref)
    ssum = s_ref[:_NH, :] + s_ref[_HPAD:_HPAD + _NH, :]
    z = jnp.maximum(dinv * (ssum + hp_ref[...]) + b2_ref[...], 0.0)
    labels = lax.broadcasted_iota(jnp.int32, (1, _B), 1)
    oh0 = (batch_ref[:, 0:1] == labels).astype(jnp.float32)
    oh1 = (batch_ref[:, 1:2] == labels).astype(jnp.float32)
    ones_col = jnp.ones((_NH, 1), jnp.float32)
    zz0 = jnp.concatenate([z[:, :_F], ones_col], axis=1)
    zz1 = jnp.concatenate([z[:, _F:], ones_col], axis=1)
    dn = (((0,), (0,)), ((), ()))
    pooled_all = (
        lax.dot_general(oh0, zz0, dn, preferred_element_type=jnp.float32)
        + lax.dot_general(oh1, zz1, dn, preferred_element_type=jnp.float32))
    sums = pooled_all[:, :_F]
    counts = pooled_all[:, _F:_F + 1]
    pooled = sums / jnp.maximum(counts, 1.0)
    logits = jnp.dot(pooled, wfc_ref[...],
                     preferred_element_type=jnp.float32) + bfc_ref[...]
    m = jnp.max(logits, axis=1, keepdims=True)
    lse = jnp.log(jnp.sum(jnp.exp(logits - m), axis=1, keepdims=True)) + m
    o_ref[...] = logits - lse


def _block_diag2(w):
    z = jnp.zeros_like(w)
    return jnp.concatenate(
        [jnp.concatenate([w, z], axis=1), jnp.concatenate([z, w], axis=1)],
        axis=0)


def kernel(x, edge_index, batch, W1, b1, W2, b2, Wfc, bfc):
    f32 = jnp.float32
    src, dst = pl.pallas_call(
        _edge_body,
        in_specs=[pl.BlockSpec(memory_space=pltpu.HBM)],
        out_specs=[pl.BlockSpec(memory_space=pltpu.HBM),
                   pl.BlockSpec(memory_space=pltpu.HBM)],
        out_shape=[jax.ShapeDtypeStruct((_E,), jnp.int32),
                   jax.ShapeDtypeStruct((_E,), jnp.int32)],
        scratch_shapes=[pltpu.SemaphoreType.DMA, pltpu.SemaphoreType.DMA],
    )(edge_index)
    zeros1 = jnp.zeros((_NPAD,), f32)
    zeros2 = jnp.zeros((_NPAD, _F), f32)

    degp = _sc_degree(dst, zeros1)
    h1p = pl.pallas_call(
        _mm_body,
        out_shape=jax.ShapeDtypeStruct((_NH, 2 * _F), f32),
    )(x.reshape(_NH, 2 * x.shape[1]), _block_diag2(W1))

    deg160 = degp.reshape(_NC * _NPAD // 128, 128)
    dinv80 = lax.rsqrt(
        deg160[:_NPAD // 128] + deg160[_NPAD // 128:] + 1.0)
    dinv2 = dinv80.reshape(_NPAD)[:_N].reshape(_NH, 2)
    hp1p = pl.pallas_call(
        _scale_body,
        out_shape=jax.ShapeDtypeStruct((_NH, 2 * _F), f32),
    )(h1p, dinv2)

    s1p = _sc_scatter(hp1p.reshape(_N, _F), src, dst, zeros2)
    hp2p = pl.pallas_call(
        _mid_body,
        out_shape=jax.ShapeDtypeStruct((_NH, 2 * _F), f32),
    )(s1p.reshape(_NC * _HPAD, 2 * _F), hp1p, dinv2, _block_diag2(W2),
      jnp.concatenate([b1, b1]).reshape(1, 2 * _F))

    s2p = _sc_scatter(hp2p.reshape(_N, _F), src, dst, zeros2)
    out = pl.pallas_call(
        _final_body,
        out_shape=jax.ShapeDtypeStruct((_B, _NCLS), f32),
    )(s2p.reshape(_NC * _HPAD, 2 * _F), hp2p, dinv2,
      jnp.concatenate([b2, b2]).reshape(1, 2 * _F),
      batch.reshape(_NH, 2), Wfc, bfc.reshape(1, _NCLS))
    return out


# speedup vs baseline: 1.3082x; 1.3082x over previous
import functools

import jax
import jax.numpy as jnp
from jax import lax
from jax.experimental import pallas as pl
from jax.experimental.pallas import tpu as pltpu
from jax.experimental.pallas import tpu_sc as plsc

_NC = 2
_NS = 16
_NW = _NC * _NS

_N = 10000
_NPAD = 10240
_RPT = _NPAD // _NS
_F = 64
_E = 320000
_EPW = _E // _NW
_C = 80
_NCH = _EPW // _C
_B = 64
_NCLS = 10

_mesh = plsc.VectorSubcoreMesh(core_axis_name="c", subcore_axis_name="s")
_cp = pltpu.CompilerParams(use_tc_tiling_on_sc=False)


_NBUF = 5


@functools.partial(
    pl.kernel,
    out_type=jax.ShapeDtypeStruct((_NC * _NPAD,), jnp.float32),
    mesh=_mesh,
    scratch_types=[
        pltpu.VMEM((_NBUF, _C), jnp.int32),
        pltpu.VMEM((_C,), jnp.float32),
        pltpu.VMEM_SHARED((_NPAD,), jnp.float32),
        pltpu.SemaphoreType.DMA((_NBUF,)),
        pltpu.SemaphoreType.DMA((_NBUF,)),
    ],
    compiler_params=_cp,
)
def _sc_degree(dst_hbm, zeros1_hbm, out_hbm, didx, ones_v, accum, isem, ssem):
    cid = lax.axis_index("c")
    sid = lax.axis_index("s")
    wid = cid * _NS + sid
    base = wid * _EPW

    @pl.loop(0, _C, step=16)
    def _(i):
        ones_v[pl.ds(i, 16)] = jnp.full((16,), 1.0, jnp.float32)

    pltpu.sync_copy(zeros1_hbm.at[pl.ds(sid * _RPT, _RPT)],
                    accum.at[pl.ds(sid * _RPT, _RPT)])
    plsc.subcore_barrier()

    def start_idx(j, b):
        pltpu.async_copy(dst_hbm.at[pl.ds(base + j * _C, _C)], didx.at[b],
                         isem.at[b])

    def wait_idx(j, b):
        pltpu.make_async_copy(dst_hbm.at[pl.ds(base + j * _C, _C)], didx.at[b],
                              isem.at[b]).wait()

    def start_scat(b):
        pltpu.async_copy(ones_v, accum.at[didx.at[b]], ssem.at[b], add=True)

    def wait_scat(b):
        pltpu.make_async_copy(ones_v, accum.at[didx.at[b]], ssem.at[b]).wait()

    for b in range(_NBUF):
        start_idx(b, b)

    @pl.loop(0, _NCH - _NBUF, step=_NBUF)
    def _(j0):
        for b in range(_NBUF):
            wait_idx(j0 + b, b)
            start_scat(b)
        for b in range(_NBUF):
            wait_scat(b)
            start_idx(j0 + _NBUF + b, b)

    for b in range(_NBUF):
        wait_idx(_NCH - _NBUF + b, b)
        start_scat(b)
    for b in range(_NBUF):
        wait_scat(b)

    plsc.subcore_barrier()
    pltpu.sync_copy(accum.at[pl.ds(sid * _RPT, _RPT)],
                    out_hbm.at[pl.ds(cid * _NPAD + sid * _RPT, _RPT)])


@functools.partial(
    pl.kernel,
    out_type=jax.ShapeDtypeStruct((_NC * _NPAD, _F), jnp.float32),
    mesh=_mesh,
    scratch_types=[
        pltpu.VMEM((_EPW,), jnp.int32),
        pltpu.VMEM((_NBUF, _C), jnp.int32),
        pltpu.VMEM((_NBUF, _C, _F), jnp.float32),
        pltpu.VMEM_SHARED((_NPAD, _F), jnp.float32),
        pltpu.SemaphoreType.DMA((_NBUF,)),
        pltpu.SemaphoreType.DMA((_NBUF,)),
        pltpu.SemaphoreType.DMA((_NBUF,)),
    ],
    compiler_params=_cp,
)
def _sc_scatter(hp_hbm, src_hbm, dst_hbm, zeros2_hbm, out_hbm,
                sidx, didx, rows, accum, gsem, isem, ssem):
    cid = lax.axis_index("c")
    sid = lax.axis_index("s")
    wid = cid * _NS + sid
    base = wid * _EPW

    pltpu.sync_copy(src_hbm.at[pl.ds(base, _EPW)], sidx)
    pltpu.sync_copy(zeros2_hbm.at[pl.ds(sid * _RPT, _RPT)],
                    accum.at[pl.ds(sid * _RPT, _RPT)])
    plsc.subcore_barrier()

    def start_idx(j, b):
        pltpu.async_copy(dst_hbm.at[pl.ds(base + j * _C, _C)], didx.at[b],
                         isem.at[b])

    def wait_idx(j, b):
        pltpu.make_async_copy(dst_hbm.at[pl.ds(base + j * _C, _C)], didx.at[b],
                              isem.at[b]).wait()

    def start_gat(j, b):
        pltpu.async_copy(hp_hbm.at[sidx.at[pl.ds(j * _C, _C)]], rows.at[b],
                         gsem.at[b])

    def wait_gat(j, b):
        pltpu.make_async_copy(hp_hbm.at[sidx.at[pl.ds(j * _C, _C)]],
                              rows.at[b], gsem.at[b]).wait()

    def start_scat(b):
        pltpu.async_copy(rows.at[b], accum.at[didx.at[b]], ssem.at[b],
                         add=True)

    def wait_scat(b):
        pltpu.make_async_copy(rows.at[b], accum.at[didx.at[b]],
                              ssem.at[b]).wait()

    for b in range(_NBUF):
        start_idx(b, b)
        start_gat(b, b)

    @pl.loop(0, _NCH - _NBUF, step=_NBUF)
    def _(j0):
        for b in range(_NBUF):
            wait_idx(j0 + b, b)
            wait_gat(j0 + b, b)
            start_scat(b)
        for b in range(_NBUF):
            wait_scat(b)
            start_idx(j0 + _NBUF + b, b)
            start_gat(j0 + _NBUF + b, b)

    for b in range(_NBUF):
        wait_idx(_NCH - _NBUF + b, b)
        wait_gat(_NCH - _NBUF + b, b)
        start_scat(b)
    for b in range(_NBUF):
        wait_scat(b)

    plsc.subcore_barrier()
    pltpu.sync_copy(accum.at[pl.ds(sid * _RPT, _RPT)],
                    out_hbm.at[pl.ds(cid * _NPAD + sid * _RPT, _RPT)])


_NH = _N // 2
_HPAD = _NPAD // 2


def _mm_body(x_ref, w_ref, o_ref):
    o_ref[...] = jnp.dot(x_ref[...], w_ref[...],
                         preferred_element_type=jnp.float32)


def _dfull(dinv2_ref):
    d0 = jnp.broadcast_to(dinv2_ref[:, 0:1], (_NH, _F))
    d1 = jnp.broadcast_to(dinv2_ref[:, 1:2], (_NH, _F))
    return jnp.concatenate([d0, d1], axis=1)


def _scale_body(h_ref, dinv2_ref, o_ref):
    o_ref[...] = _dfull(dinv2_ref) * h_ref[...]


def _mid_body(s_ref, hp_ref, dinv2_ref, w2_ref, b1_ref, h2p_ref):
    dinv = _dfull(dinv2_ref)
    ssum = s_ref[:_NH, :] + s_ref[_HPAD:_HPAD + _NH, :]
    z = jnp.maximum(dinv * (ssum + hp_ref[...]) + b1_ref[...], 0.0)
    h2 = jnp.dot(z, w2_ref[...], preferred_element_type=jnp.float32)
    h2p_ref[...] = dinv * h2


def _final_body(s_ref, hp_ref, dinv2_ref, b2_ref, batch_ref,
                wfc_ref, bfc_ref, o_ref):
    dinv = _dfull(dinv2_ref)
    ssum = s_ref[:_NH, :] + s_ref[_HPAD:_HPAD + _NH, :]
    z = jnp.maximum(dinv * (ssum + hp_ref[...]) + b2_ref[...], 0.0)
    labels = lax.broadcasted_iota(jnp.int32, (1, _B), 1)
    oh0 = (batch_ref[:, 0:1] == labels).astype(jnp.float32)
    oh1 = (batch_ref[:, 1:2] == labels).astype(jnp.float32)
    ones_col = jnp.ones((_NH, 1), jnp.float32)
    zz0 = jnp.concatenate([z[:, :_F], ones_col], axis=1)
    zz1 = jnp.concatenate([z[:, _F:], ones_col], axis=1)
    dn = (((0,), (0,)), ((), ()))
    pooled_all = (
        lax.dot_general(oh0, zz0, dn, preferred_element_type=jnp.float32)
        + lax.dot_general(oh1, zz1, dn, preferred_element_type=jnp.float32))
    sums = pooled_all[:, :_F]
    counts = pooled_all[:, _F:_F + 1]
    pooled = sums / jnp.maximum(counts, 1.0)
    logits = jnp.dot(pooled, wfc_ref[...],
                     preferred_element_type=jnp.float32) + bfc_ref[...]
    m = jnp.max(logits, axis=1, keepdims=True)
    lse = jnp.log(jnp.sum(jnp.exp(logits - m), axis=1, keepdims=True)) + m
    o_ref[...] = logits - lse


def _block_diag2(w):
    z = jnp.zeros_like(w)
    return jnp.concatenate(
        [jnp.concatenate([w, z], axis=1), jnp.concatenate([z, w], axis=1)],
        axis=0)


def kernel(x, edge_index, batch, W1, b1, W2, b2, Wfc, bfc):
    f32 = jnp.float32
    src = edge_index[0]
    dst = edge_index[1]
    zeros1 = jnp.zeros((_NPAD,), f32)
    zeros2 = jnp.zeros((_NPAD, _F), f32)

    degp = _sc_degree(dst, zeros1)
    h1p = pl.pallas_call(
        _mm_body,
        out_shape=jax.ShapeDtypeStruct((_NH, 2 * _F), f32),
    )(x.reshape(_NH, 2 * x.shape[1]), _block_diag2(W1))

    deg160 = degp.reshape(_NC * _NPAD // 128, 128)
    dinv80 = lax.rsqrt(
        deg160[:_NPAD // 128] + deg160[_NPAD // 128:] + 1.0)
    dinv2 = dinv80.reshape(_NPAD)[:_N].reshape(_NH, 2)
    hp1p = pl.pallas_call(
        _scale_body,
        out_shape=jax.ShapeDtypeStruct((_NH, 2 * _F), f32),
    )(h1p, dinv2)

    s1p = _sc_scatter(hp1p.reshape(_N, _F), src, dst, zeros2)
    hp2p = pl.pallas_call(
        _mid_body,
        out_shape=jax.ShapeDtypeStruct((_NH, 2 * _F), f32),
    )(s1p.reshape(_NC * _HPAD, 2 * _F), hp1p, dinv2, _block_diag2(W2),
      jnp.concatenate([b1, b1]).reshape(1, 2 * _F))

    s2p = _sc_scatter(hp2p.reshape(_N, _F), src, dst, zeros2)
    out = pl.pallas_call(
        _final_body,
        out_shape=jax.ShapeDtypeStruct((_B, _NCLS), f32),
    )(s2p.reshape(_NC * _HPAD, 2 * _F), hp2p, dinv2,
      jnp.concatenate([b2, b2]).reshape(1, 2 * _F),
      batch.reshape(_NH, 2), Wfc, bfc.reshape(1, _NCLS))
    return out
